# Initial kernel scaffold; baseline (speedup 1.0000x reference)
import jax
import jax.numpy as jnp
from jax.experimental import pallas as pl


def kernel(x, edge_index, h_msg, y_msg, W_in, b_in, W_enc, b_enc, W_m1, b_m1, W_m2, b_m2, W_u, b_u, W_dec, b_dec):
    raise NotImplementedError("write your pallas kernel here")


# speedup vs baseline: 2.3969x; 2.3969x over previous
import functools

import jax
import jax.numpy as jnp
from jax import lax
from jax.experimental import pallas as pl
from jax.experimental.pallas import tpu as pltpu
from jax.experimental.pallas import tpu_sc as plsc

_N = 100000
_E = 1600000
_H = 32
_MSG = 2

_NC = 2
_NS = 16
_NW = _NC * _NS
_EW = _E // _NW
_SG = 125
_NSG = 8
_CH = _SG * _NSG
_CPW = _EW // _CH
_ROWS = _E // _SG


def _node_tables_body(x_ref, wa_ref, wb_ref, ca_ref, cb_ref, a_ref, b_ref):
    xb = x_ref[...]
    a_ref[...] = (
        jnp.dot(xb, wa_ref[...], preferred_element_type=jnp.float32) + ca_ref[...]
    )
    b_ref[...] = (
        jnp.dot(xb, wb_ref[...], preferred_element_type=jnp.float32) + cb_ref[...]
    )


def _node_tables(x, wa_t, wb_t, ca, cb):
    bn = 10000
    grid = (_N // bn,)
    return pl.pallas_call(
        _node_tables_body,
        grid=grid,
        in_specs=[
            pl.BlockSpec((bn, 2), lambda i: (i, 0)),
            pl.BlockSpec((2, _H), lambda i: (0, 0)),
            pl.BlockSpec((2, _H), lambda i: (0, 0)),
            pl.BlockSpec((1, _H), lambda i: (0, 0)),
            pl.BlockSpec((1, _H), lambda i: (0, 0)),
        ],
        out_specs=[
            pl.BlockSpec((bn, _H), lambda i: (i, 0)),
            pl.BlockSpec((bn, _H), lambda i: (i, 0)),
        ],
        out_shape=[
            jax.ShapeDtypeStruct((_N, _H), jnp.float32),
            jax.ShapeDtypeStruct((_N, _H), jnp.float32),
        ],
    )(x, wa_t, wb_t, ca, cb)


def _gather_body(a_hbm, b_hbm, dst_hbm, src_hbm, g_hbm, dstv, srcv, rows, sem_g):
    c = lax.axis_index("c")
    s = lax.axis_index("s")
    wid = s * _NC + c
    base_row = wid * (_EW // _SG)

    def chunk_body(k, carry):
        row0 = base_row + k * _NSG
        pltpu.sync_copy(dst_hbm.at[pl.ds(row0, _NSG)], dstv)
        pltpu.sync_copy(src_hbm.at[pl.ds(row0, _NSG)], srcv)

        def fire_a(j, cy):
            pltpu.async_copy(a_hbm.at[dstv.at[j]], rows.at[j], sem_g)
            return cy

        lax.fori_loop(0, _NSG, fire_a, 0)

        def drain_a(j, cy):
            pltpu.make_async_copy(a_hbm.at[dstv.at[j]], rows.at[j], sem_g).wait()
            return cy

        lax.fori_loop(0, _NSG, drain_a, 0)

        def fire_b(j, cy):
            pltpu.async_copy(b_hbm.at[srcv.at[j]], rows.at[j], sem_g, add=True)
            return cy

        lax.fori_loop(0, _NSG, fire_b, 0)

        def drain_b(j, cy):
            pltpu.make_async_copy(b_hbm.at[srcv.at[j]], rows.at[j], sem_g).wait()
            return cy

        lax.fori_loop(0, _NSG, drain_b, 0)

        pltpu.sync_copy(rows, g_hbm.at[pl.ds(row0, _NSG)])
        return carry

    lax.fori_loop(0, _CPW, chunk_body, 0)


def _gather_add(a, b, dst3, src3):
    mesh = plsc.VectorSubcoreMesh(
        core_axis_name="c", subcore_axis_name="s", num_cores=_NC, num_subcores=_NS
    )
    fn = pl.kernel(
        _gather_body,
        out_type=jax.ShapeDtypeStruct((_ROWS, _SG, _H), jnp.float32),
        mesh=mesh,
        scratch_types=[
            pltpu.VMEM((_NSG, _SG), jnp.int32),
            pltpu.VMEM((_NSG, _SG), jnp.int32),
            pltpu.VMEM((_NSG, _SG, _H), jnp.float32),
            pltpu.SemaphoreType.DMA,
        ],
        compiler_params=pltpu.CompilerParams(use_tc_tiling_on_sc=False),
    )
    return fn(a, b, dst3, src3)


def _mlp_body(
    g_ref, y_ref, h_ref, wc_ref, wm2_ref, bm2_ref, wdec_ref, bdec_ref,
    hnew_ref, ynew_ref,
):
    yh = jnp.concatenate([y_ref[...], h_ref[...]], axis=1)
    m1 = g_ref[...] + jnp.dot(yh, wc_ref[...], preferred_element_type=jnp.float32)
    m1 = jnp.where(m1 > 0, m1, 0.01 * m1)
    m2 = jnp.dot(m1, wm2_ref[...], preferred_element_type=jnp.float32) + bm2_ref[...]
    m2 = jnp.where(m2 > 0, m2, 0.01 * m2)
    hnew_ref[...] = m2
    lg = jnp.dot(m2, wdec_ref[...], preferred_element_type=jnp.float32) + bdec_ref[...]
    mx = jnp.max(lg, axis=1, keepdims=True)
    ex = jnp.exp(lg - mx)
    ynew_ref[...] = ex / jnp.sum(ex, axis=1, keepdims=True)


def _edge_mlp(g, y_msg, h_msg, wc_t, wm2_t, bm2, wdec_t, bdec):
    be = 6400
    grid = (_E // be,)
    return pl.pallas_call(
        _mlp_body,
        grid=grid,
        in_specs=[
            pl.BlockSpec((be, _H), lambda i: (i, 0)),
            pl.BlockSpec((be, _MSG), lambda i: (i, 0)),
            pl.BlockSpec((be, _H), lambda i: (i, 0)),
            pl.BlockSpec((_MSG + _H, _H), lambda i: (0, 0)),
            pl.BlockSpec((_H, _H), lambda i: (0, 0)),
            pl.BlockSpec((1, _H), lambda i: (0, 0)),
            pl.BlockSpec((_H, _MSG), lambda i: (0, 0)),
            pl.BlockSpec((1, _MSG), lambda i: (0, 0)),
        ],
        out_specs=[
            pl.BlockSpec((be, _H), lambda i: (i, 0)),
            pl.BlockSpec((be, _MSG), lambda i: (i, 0)),
        ],
        out_shape=[
            jax.ShapeDtypeStruct((_E, _H), jnp.float32),
            jax.ShapeDtypeStruct((_E, _MSG), jnp.float32),
        ],
    )(g, y_msg, h_msg, wc_t, wm2_t, bm2, wdec_t, bdec)


def kernel(x, edge_index, h_msg, y_msg, W_in, b_in, W_enc, b_enc, W_m1, b_m1,
           W_m2, b_m2, W_u, b_u, W_dec, b_dec):
    w_m1a = W_m1[:, :_H]
    w_m1b = W_m1[:, _H:2 * _H]
    w_m1c = W_m1[:, 2 * _H:]
    wa = w_m1a @ W_in
    wb = w_m1b @ W_in
    wc = w_m1c @ W_enc
    ca = w_m1a @ b_in + w_m1c @ b_enc + b_m1
    cb = w_m1b @ b_in

    a, b = _node_tables(x, wa.T, wb.T, ca[None, :], cb[None, :])

    dst3 = edge_index[1].reshape(_ROWS, _SG)
    src3 = edge_index[0].reshape(_ROWS, _SG)
    g3 = _gather_add(a, b, dst3, src3)
    g = g3.reshape(_E, _H)

    hnew, ynew = _edge_mlp(
        g, y_msg, h_msg, wc.T, W_m2.T, b_m2[None, :], W_dec.T, b_dec[None, :]
    )
    return hnew, ynew
